# Initial kernel scaffold; baseline (speedup 1.0000x reference)
import jax
import jax.numpy as jnp
from jax.experimental import pallas as pl


def kernel(x, W1, b1, W2, b2, W3, b3, Wc, bc, edge_index):
    raise NotImplementedError("write your pallas kernel here")


# speedup vs baseline: 6.4889x; 6.4889x over previous
import functools

import jax
import jax.numpy as jnp
from jax import lax
from jax.experimental import pallas as pl
from jax.experimental.pallas import tpu as pltpu
from jax.experimental.pallas import tpu_sc as plsc

N = 10000
E = 320000
DA = 128
NC = 2
NS = 16
NW = NC * NS
EPW = E // NW
K = 80
NCH = EPW // K
RPT = 624
TAIL = N - NS * RPT

_EPS = 1e-7
_ACOSH_LO = 1.0 + 1e-7


def _row_norm(t):
    return jnp.maximum(jnp.sqrt(jnp.sum(t * t, axis=-1, keepdims=True)), _EPS)


def _acosh(w):
    w = jnp.maximum(w, _ACOSH_LO)
    return jnp.log(w + jnp.sqrt(w * w - 1.0))


def _f(t):
    n = _row_norm(t)
    en = jnp.exp(n)
    eninv = jnp.exp(-n)
    coshn = 0.5 * (en + eninv)
    sinhn = 0.5 * (en - eninv)
    xs = (sinhn / n) * t
    xsn = _row_norm(xs)
    d = _acosh(coshn)
    return (d / xsn) * xs


def _ulog_body(x_ref, o_ref):
    x = x_ref[...]
    col = lax.broadcasted_iota(jnp.int32, x.shape, 1)
    xs = jnp.where(col >= 1, x, 0.0)
    xsn = _row_norm(xs)
    d = _acosh(x[:, 0:1])
    o_ref[...] = (d / xsn) * xs


def _ulog(x):
    bu = 2000
    return pl.pallas_call(
        _ulog_body,
        grid=(N // bu,),
        in_specs=[pl.BlockSpec((bu, DA), lambda i: (i, 0))],
        out_specs=pl.BlockSpec((bu, DA), lambda i: (i, 0)),
        out_shape=jax.ShapeDtypeStruct((N, DA), jnp.float32),
    )(x)


def _sc_agg_body(u_hbm, src_hbm, dst_hbm, zeros_hbm, out_hbm,
                 src_v, dst_v, rows_v, agg_sh, gsem):
    c = lax.axis_index("c")
    s = lax.axis_index("s")
    wid = s * NC + c
    pltpu.sync_copy(src_hbm.at[wid], src_v)
    pltpu.sync_copy(dst_hbm.at[wid], dst_v)
    pltpu.sync_copy(zeros_hbm, agg_sh.at[pl.ds(s * RPT, RPT)])

    @pl.when(s == NS - 1)
    def _():
        pltpu.sync_copy(zeros_hbm.at[pl.ds(0, TAIL)],
                        agg_sh.at[pl.ds(NS * RPT, TAIL)])

    plsc.subcore_barrier()

    def chunk(j, carry):
        pltpu.async_copy(u_hbm.at[src_v.at[j]], rows_v, gsem).wait()
        pltpu.sync_copy(rows_v, agg_sh.at[dst_v.at[j]], add=True)
        return carry

    lax.fori_loop(0, NCH, chunk, 0)
    plsc.subcore_barrier()
    pltpu.sync_copy(agg_sh.at[pl.ds(s * RPT, RPT)],
                    out_hbm.at[c, pl.ds(s * RPT, RPT)])

    @pl.when(s == NS - 1)
    def _():
        pltpu.sync_copy(agg_sh.at[pl.ds(NS * RPT, TAIL)],
                        out_hbm.at[c, pl.ds(NS * RPT, TAIL)])


def _sc_agg(u, src_r, dst_r, zeros):
    mesh = plsc.VectorSubcoreMesh(core_axis_name="c", subcore_axis_name="s")
    kern = functools.partial(
        pl.kernel,
        mesh=mesh,
        out_type=jax.ShapeDtypeStruct((NC, N, DA), jnp.float32),
        scratch_types=[
            pltpu.VMEM((NCH, K), jnp.int32),
            pltpu.VMEM((NCH, K), jnp.int32),
            pltpu.VMEM((K, DA), jnp.float32),
            pltpu.VMEM_SHARED((N, DA), jnp.float32),
            pltpu.SemaphoreType.DMA,
        ],
    )(_sc_agg_body)
    return kern(u, src_r, dst_r, zeros)


def _chain_body(u_ref, a0_ref, a1_ref, w1_ref, b1_ref, w2_ref, b2_ref,
                w3_ref, b3_ref, acc_ref):
    z = u_ref[...] + a0_ref[...] + a1_ref[...]
    s0 = _f(z)
    t1 = jnp.dot(s0, w1_ref[...], preferred_element_type=jnp.float32) + b1_ref[...]
    g1 = _f(jnp.maximum(_f(t1), 0.0))
    t2 = jnp.dot(g1, w2_ref[...], preferred_element_type=jnp.float32) + b2_ref[...]
    g2 = _f(jnp.maximum(_f(t2), 0.0))
    t3 = jnp.dot(g2, w3_ref[...], preferred_element_type=jnp.float32) + b3_ref[...]
    pool = _f(jnp.maximum(_f(t3), 0.0))
    psum = jnp.sum(pool, axis=0, keepdims=True)

    @pl.when(pl.program_id(0) == 0)
    def _():
        acc_ref[...] = jnp.zeros_like(acc_ref)

    acc_ref[...] += psum


def _chain(u, a0, a1, w1p, b1p, w2p, b2p, w3p, b3p):
    b = 400
    full = lambda i: (0, 0)
    return pl.pallas_call(
        _chain_body,
        grid=(N // b,),
        in_specs=[
            pl.BlockSpec((b, DA), lambda i: (i, 0)),
            pl.BlockSpec((b, DA), lambda i: (i, 0)),
            pl.BlockSpec((b, DA), lambda i: (i, 0)),
            pl.BlockSpec((128, 128), full),
            pl.BlockSpec((1, 128), full),
            pl.BlockSpec((128, 256), full),
            pl.BlockSpec((1, 256), full),
            pl.BlockSpec((256, 384), full),
            pl.BlockSpec((1, 384), full),
        ],
        out_specs=pl.BlockSpec((1, 384), full),
        out_shape=jax.ShapeDtypeStruct((1, 384), jnp.float32),
    )(u, a0, a1, w1p, b1p, w2p, b2p, w3p, b3p)


def _tail_body(acc_ref, wc_ref, bc_ref, hc_ref, prob_ref):
    mean = acc_ref[...] * (1.0 / N)
    g = _f(mean)
    tc = jnp.dot(g, wc_ref[...], preferred_element_type=jnp.float32) + bc_ref[...]
    col = lax.broadcasted_iota(jnp.int32, tc.shape, 1)
    valid = jnp.logical_and(col >= 1, col <= 9)

    n = _row_norm(tc)
    en = jnp.exp(n)
    eninv = jnp.exp(-n)
    coshn = 0.5 * (en + eninv)
    sinhn = 0.5 * (en - eninv)
    xs = (sinhn / n) * tc
    hc = jnp.where(col == 0, coshn, xs)
    hc_ref[...] = hc

    xsn = _row_norm(xs)
    d = _acosh(coshn)
    gg = (d / xsn) * xs

    gmax = jnp.max(jnp.where(valid, gg, -jnp.inf), axis=-1, keepdims=True)
    e = jnp.where(valid, jnp.exp(gg - gmax), 0.0)
    sm = e / jnp.sum(e, axis=-1, keepdims=True)

    n2 = _row_norm(sm)
    en2 = jnp.exp(n2)
    en2inv = jnp.exp(-n2)
    cosh2 = 0.5 * (en2 + en2inv)
    sinh2 = 0.5 * (en2 - en2inv)
    xs2 = (sinh2 / n2) * sm
    prob_ref[...] = jnp.where(col == 0, cosh2, xs2)


def _tail(acc, wcp, bcp):
    return pl.pallas_call(
        _tail_body,
        out_shape=(jax.ShapeDtypeStruct((1, 128), jnp.float32),
                   jax.ShapeDtypeStruct((1, 128), jnp.float32)),
    )(acc, wcp, bcp)


def kernel(x, W1, b1, W2, b2, W3, b3, Wc, bc, edge_index):
    w1p = jnp.zeros((128, 128), jnp.float32).at[1:, 1:].set(W1)
    b1p = jnp.zeros((1, 128), jnp.float32).at[0, 1:].set(b1)
    w2p = jnp.zeros((128, 256), jnp.float32).at[1:, 1:].set(W2)
    b2p = jnp.zeros((1, 256), jnp.float32).at[0, 1:].set(b2)
    w3p = jnp.zeros((256, 384), jnp.float32).at[1:, 1:].set(W3)
    b3p = jnp.zeros((1, 384), jnp.float32).at[0, 1:].set(b3)
    wcp = jnp.zeros((384, 128), jnp.float32).at[1:384, 1:10].set(Wc)
    bcp = jnp.zeros((1, 128), jnp.float32).at[0, 1:10].set(bc)

    src_r = edge_index[0].reshape(NW, NCH, K)
    dst_r = edge_index[1].reshape(NW, NCH, K)
    zeros = jnp.zeros((RPT, DA), jnp.float32)

    u = _ulog(x)
    aggs = _sc_agg(u, src_r, dst_r, zeros)
    acc = _chain(u, aggs[0], aggs[1], w1p, b1p, w2p, b2p, w3p, b3p)
    hc, prob = _tail(acc, wcp, bcp)
    return (hc[0, :10], prob[0, :10])
